# baseline (speedup 1.0000x reference)
import functools

import jax
import jax.numpy as jnp
from jax import lax
from jax.experimental import pallas as pl
from jax.experimental.pallas import tpu as pltpu
from jax.experimental.pallas import tpu_sc as plsc

_NW = 32
_CB = 128
_NCH = 25
_PW = _CB * _NCH
_NP = _NW * _PW
_BN2 = 2048
_NBLK2 = _NP // _BN2
_BN3 = 512


def _min_body(c_ref, o_ref):
    o_ref[...] = jnp.min(c_ref[...], axis=0, keepdims=True)


def _proj_body(shift_ref, lm_ref, intr_ref, xs_ref, ys_ref, zs_ref,
               pk_ref, cnt_ref, *, H, W, HW):
    sx = shift_ref[0, 1]
    sy = shift_ref[0, 2]
    sz = shift_ref[0, 3]
    lx = (xs_ref[0, 0, :] - sx).astype(jnp.float32) + 0.5
    ly = (ys_ref[0, 0, :] - sy).astype(jnp.float32) + 0.5
    lz = (zs_ref[0, 0, :] - sz).astype(jnp.float32) + 0.5
    fx = intr_ref[0, 0]
    fy = intr_ref[0, 1]
    cx = intr_ref[0, 2]
    cy = intr_ref[0, 3]
    cnt = jnp.zeros(lx.shape, jnp.int32)
    for v in range(4):
        cam0 = lx * lm_ref[3 * v + 0, 0] + ly * lm_ref[3 * v + 0, 1] \
            + lz * lm_ref[3 * v + 0, 2] + lm_ref[3 * v + 0, 3]
        cam1 = lx * lm_ref[3 * v + 1, 0] + ly * lm_ref[3 * v + 1, 1] \
            + lz * lm_ref[3 * v + 1, 2] + lm_ref[3 * v + 1, 3]
        z = lx * lm_ref[3 * v + 2, 0] + ly * lm_ref[3 * v + 2, 1] \
            + lz * lm_ref[3 * v + 2, 2] + lm_ref[3 * v + 2, 3]
        zc = jnp.where(jnp.abs(z) < 1e-6, 1e-6, z)
        u = fx * cam0 / zc + cx
        vv = fy * cam1 / zc + cy
        valid = (z >= 0.1 / 0.05) & (z <= 4.0 / 0.05) \
            & (u >= 0.0) & (u < float(W)) & (vv >= 0.0) & (vv < float(H))
        ui = jnp.floor(jnp.clip(u, 0.0, float(W - 1))).astype(jnp.int32)
        vi = jnp.floor(jnp.clip(vv, 0.0, float(H - 1))).astype(jnp.int32)
        lin = v * HW + vi * W + ui
        pk_ref[0, v, :] = jnp.where(valid, lin, 0)
        cnt = cnt + valid.astype(jnp.int32)
    cnt_ref[0, 0, :] = cnt


def _sc_body(feats_hbm, pk_hbm, out_hbm, g, sem):
    def _step(iv, ov):
        pltpu.async_copy(feats_hbm.at[iv.at[0, 0]], g, sem).wait()

        @pl.loop(0, _CB)
        def _row(r):
            for c in range(4):
                sl = pl.ds(c * 16, 16)
                a = g[r, sl] + g[_CB + r, sl]
                b = g[2 * _CB + r, sl] + g[3 * _CB + r, sl]
                ov[r, sl] = a + b

    pltpu.emit_pipeline(
        _step,
        grid=(_NP // _CB,),
        in_specs=[pl.BlockSpec((1, 1, 4 * _CB), lambda i: (i, 0, 0))],
        out_specs=[pl.BlockSpec((_CB, 64), lambda i: (i, 0))],
        core_axis_name=("c", "s"),
        dimension_semantics=(pltpu.PARALLEL,),
    )(pk_hbm, out_hbm)


def _final_body(sum_ref, cnt_ref, r0_ref, o_ref):
    c = cnt_ref[...].astype(jnp.float32)
    r0 = r0_ref[0:1, :]
    o_ref[...] = (sum_ref[...] - (4.0 - c) * r0) / (c + 1e-4)


def kernel(encoded_2d_features, coords, view_matrix, intrinsic_params):
    B, V, H, W, C = encoded_2d_features.shape
    N = coords.shape[0]
    HW = H * W
    feats_flat = encoded_2d_features.reshape(V * HW, C)

    cr = coords.reshape((N * 4) // 128, 128)
    colmin = pl.pallas_call(
        _min_body,
        out_shape=jax.ShapeDtypeStruct((1, 128), jnp.int32),
    )(cr)
    shift4 = colmin.reshape(32, 4).min(axis=0).reshape(1, 4)

    shift_f = shift4[0, 1:4].astype(jnp.float32)
    lview = view_matrix[0].at[:, :3, 3].add(-shift_f)
    lm = lview[:, :3, :].reshape(12, 4)
    intr = intrinsic_params.reshape(1, 4)
    pad = _NP - N
    xs = jnp.pad(coords[:, 1], (0, pad)).reshape(_NBLK2, 1, _BN2)
    ys = jnp.pad(coords[:, 2], (0, pad)).reshape(_NBLK2, 1, _BN2)
    zs = jnp.pad(coords[:, 3], (0, pad)).reshape(_NBLK2, 1, _BN2)

    blk = pl.BlockSpec((1, 1, _BN2), lambda i: (i, 0, 0))
    smem = pl.BlockSpec(memory_space=pltpu.SMEM)
    pk_raw, cnt = pl.pallas_call(
        functools.partial(_proj_body, H=H, W=W, HW=HW),
        grid=(_NBLK2,),
        in_specs=[smem, smem, smem, blk, blk, blk],
        out_specs=[pl.BlockSpec((1, 4, _BN2), lambda i: (i, 0, 0)), blk],
        out_shape=[
            jax.ShapeDtypeStruct((_NBLK2, 4, _BN2), jnp.int32),
            jax.ShapeDtypeStruct((_NBLK2, 1, _BN2), jnp.int32),
        ],
    )(shift4, lm, intr, xs, ys, zs)
    pk = pk_raw.reshape(_NBLK2, 4, _BN2 // _CB, _CB).transpose(0, 2, 1, 3) \
        .reshape(_NP // _CB, 1, 4 * _CB)

    mesh = plsc.VectorSubcoreMesh(core_axis_name="c", subcore_axis_name="s")
    sc = pl.kernel(
        _sc_body,
        mesh=mesh,
        compiler_params=pltpu.CompilerParams(use_tc_tiling_on_sc=False),
        out_type=jax.ShapeDtypeStruct((_NP, C), jnp.float32),
        scratch_types=[
            pltpu.VMEM((4 * _CB, C), jnp.float32),
            pltpu.SemaphoreType.DMA,
        ],
    )
    ssum = sc(feats_flat, pk)

    row0 = jnp.broadcast_to(feats_flat[0:1, :], (8, C))
    out = pl.pallas_call(
        _final_body,
        grid=(pl.cdiv(N, _BN3),),
        in_specs=[
            pl.BlockSpec((_BN3, C), lambda i: (i, 0)),
            pl.BlockSpec((_BN3, 1), lambda i: (i, 0)),
            pl.BlockSpec((8, C), lambda i: (0, 0)),
        ],
        out_specs=pl.BlockSpec((_BN3, C), lambda i: (i, 0)),
        out_shape=jax.ShapeDtypeStruct((N, C), jnp.float32),
    )(ssum, cnt.reshape(_NP, 1), row0)

    return out, cnt.reshape(_NP)[:N]


# speedup vs baseline: 8.0678x; 8.0678x over previous
import functools

import jax
import jax.numpy as jnp
from jax import lax
from jax.experimental import pallas as pl
from jax.experimental.pallas import tpu as pltpu
from jax.experimental.pallas import tpu_sc as plsc

_NW = 32
_CB = 128
_NCH = 25
_PW = _CB * _NCH
_NP = _NW * _PW
_BN2 = 2048
_NBLK2 = _NP // _BN2
_BN3 = 512
_ZPAD = 8192


def _min_body(c_ref, o_ref):
    o_ref[...] = jnp.min(c_ref[...], axis=0, keepdims=True)


def _proj_body(shift_ref, lm_ref, intr_ref, xs_ref, ys_ref, zs_ref,
               pk_ref, cnt_ref, *, H, W, HW):
    sx = shift_ref[0, 1]
    sy = shift_ref[0, 2]
    sz = shift_ref[0, 3]
    lx = (xs_ref[0, 0, :] - sx).astype(jnp.float32) + 0.5
    ly = (ys_ref[0, 0, :] - sy).astype(jnp.float32) + 0.5
    lz = (zs_ref[0, 0, :] - sz).astype(jnp.float32) + 0.5
    fx = intr_ref[0, 0]
    fy = intr_ref[0, 1]
    cx = intr_ref[0, 2]
    cy = intr_ref[0, 3]
    cnt = jnp.zeros(lx.shape, jnp.int32)
    for v in range(4):
        cam0 = lx * lm_ref[3 * v + 0, 0] + ly * lm_ref[3 * v + 0, 1] \
            + lz * lm_ref[3 * v + 0, 2] + lm_ref[3 * v + 0, 3]
        cam1 = lx * lm_ref[3 * v + 1, 0] + ly * lm_ref[3 * v + 1, 1] \
            + lz * lm_ref[3 * v + 1, 2] + lm_ref[3 * v + 1, 3]
        z = lx * lm_ref[3 * v + 2, 0] + ly * lm_ref[3 * v + 2, 1] \
            + lz * lm_ref[3 * v + 2, 2] + lm_ref[3 * v + 2, 3]
        zc = jnp.where(jnp.abs(z) < 1e-6, 1e-6, z)
        u = fx * cam0 / zc + cx
        vv = fy * cam1 / zc + cy
        valid = (z >= 0.1 / 0.05) & (z <= 4.0 / 0.05) \
            & (u >= 0.0) & (u < float(W)) & (vv >= 0.0) & (vv < float(H))
        ui = jnp.floor(jnp.clip(u, 0.0, float(W - 1))).astype(jnp.int32)
        vi = jnp.floor(jnp.clip(vv, 0.0, float(H - 1))).astype(jnp.int32)
        lin = v * HW + vi * W + ui
        lane = jax.lax.iota(jnp.int32, _BN2)
        pad_idx = 4 * HW + ((lane * 4 + v) & (_ZPAD - 1))
        pk_ref[0, v, :] = jnp.where(valid, lin, pad_idx)
        cnt = cnt + valid.astype(jnp.int32)
    cnt_ref[0, 0, :] = cnt


def _sc_body(feats_hbm, pk_hbm, out_hbm, g, sem):
    def _step(iv, ov):
        pltpu.async_copy(feats_hbm.at[iv.at[0, 0]], g, sem).wait()

        @pl.loop(0, _CB)
        def _row(r):
            for c in range(4):
                sl = pl.ds(c * 16, 16)
                a = g[r, sl] + g[_CB + r, sl]
                b = g[2 * _CB + r, sl] + g[3 * _CB + r, sl]
                ov[r, sl] = a + b

    pltpu.emit_pipeline(
        _step,
        grid=(_NP // _CB,),
        in_specs=[pl.BlockSpec((1, 1, 4 * _CB), lambda i: (i, 0, 0))],
        out_specs=[pl.BlockSpec((_CB, 64), lambda i: (i, 0))],
        core_axis_name=("c", "s"),
        dimension_semantics=(pltpu.PARALLEL,),
    )(pk_hbm, out_hbm)


def _final_body(sum_ref, cnt_ref, o_ref):
    c = cnt_ref[...].astype(jnp.float32)
    o_ref[...] = sum_ref[...] / (c + 1e-4)


def kernel(encoded_2d_features, coords, view_matrix, intrinsic_params):
    B, V, H, W, C = encoded_2d_features.shape
    N = coords.shape[0]
    HW = H * W
    feats_flat = encoded_2d_features.reshape(V * HW, C)

    cr = coords.reshape((N * 4) // 128, 128)
    colmin = pl.pallas_call(
        _min_body,
        out_shape=jax.ShapeDtypeStruct((1, 128), jnp.int32),
    )(cr)
    shift4 = colmin.reshape(32, 4).min(axis=0).reshape(1, 4)

    shift_f = shift4[0, 1:4].astype(jnp.float32)
    lview = view_matrix[0].at[:, :3, 3].add(-shift_f)
    lm = lview[:, :3, :].reshape(12, 4)
    intr = intrinsic_params.reshape(1, 4)
    pad = _NP - N
    xs = jnp.pad(coords[:, 1], (0, pad)).reshape(_NBLK2, 1, _BN2)
    ys = jnp.pad(coords[:, 2], (0, pad)).reshape(_NBLK2, 1, _BN2)
    zs = jnp.pad(coords[:, 3], (0, pad)).reshape(_NBLK2, 1, _BN2)

    blk = pl.BlockSpec((1, 1, _BN2), lambda i: (i, 0, 0))
    smem = pl.BlockSpec(memory_space=pltpu.SMEM)
    pk_raw, cnt = pl.pallas_call(
        functools.partial(_proj_body, H=H, W=W, HW=HW),
        grid=(_NBLK2,),
        in_specs=[smem, smem, smem, blk, blk, blk],
        out_specs=[pl.BlockSpec((1, 4, _BN2), lambda i: (i, 0, 0)), blk],
        out_shape=[
            jax.ShapeDtypeStruct((_NBLK2, 4, _BN2), jnp.int32),
            jax.ShapeDtypeStruct((_NBLK2, 1, _BN2), jnp.int32),
        ],
    )(shift4, lm, intr, xs, ys, zs)
    pk = pk_raw.reshape(_NBLK2, 4, _BN2 // _CB, _CB).transpose(0, 2, 1, 3) \
        .reshape(_NP // _CB, 1, 4 * _CB)

    mesh = plsc.VectorSubcoreMesh(core_axis_name="c", subcore_axis_name="s")
    sc = pl.kernel(
        _sc_body,
        mesh=mesh,
        compiler_params=pltpu.CompilerParams(use_tc_tiling_on_sc=False),
        out_type=jax.ShapeDtypeStruct((_NP, C), jnp.float32),
        scratch_types=[
            pltpu.VMEM((4 * _CB, C), jnp.float32),
            pltpu.SemaphoreType.DMA,
        ],
    )
    feats_z = jnp.concatenate(
        [feats_flat, jnp.zeros((_ZPAD, C), jnp.float32)], axis=0)
    ssum = sc(feats_z, pk)

    out = pl.pallas_call(
        _final_body,
        grid=(pl.cdiv(N, _BN3),),
        in_specs=[
            pl.BlockSpec((_BN3, C), lambda i: (i, 0)),
            pl.BlockSpec((_BN3, 1), lambda i: (i, 0)),
        ],
        out_specs=pl.BlockSpec((_BN3, C), lambda i: (i, 0)),
        out_shape=jax.ShapeDtypeStruct((N, C), jnp.float32),
    )(ssum, cnt.reshape(_NP, 1))

    return out, cnt.reshape(_NP)[:N]


# speedup vs baseline: 11.9026x; 1.4753x over previous
import functools

import jax
import jax.numpy as jnp
from jax import lax
from jax.experimental import pallas as pl
from jax.experimental.pallas import tpu as pltpu
from jax.experimental.pallas import tpu_sc as plsc

_NW = 32
_CB = 128
_NCH = 25
_PW = _CB * _NCH
_NP = _NW * _PW
_BN2 = 2048
_NBLK2 = _NP // _BN2
_ZPAD = 8192
_BIG = 1 << 29


def _min_body(x_ref, y_ref, z_ref, ox_ref, oy_ref, oz_ref):
    ox_ref[...] = jnp.min(x_ref[...], axis=0, keepdims=True)
    oy_ref[...] = jnp.min(y_ref[...], axis=0, keepdims=True)
    oz_ref[...] = jnp.min(z_ref[...], axis=0, keepdims=True)


def _proj_body(shift_ref, lm_ref, intr_ref, xs_ref, ys_ref, zs_ref,
               pk_ref, cnt_ref, rc_ref, *, H, W, HW):
    sx = shift_ref[0, 1]
    sy = shift_ref[0, 2]
    sz = shift_ref[0, 3]
    lx = (xs_ref[0, 0, :] - sx).astype(jnp.float32) + 0.5
    ly = (ys_ref[0, 0, :] - sy).astype(jnp.float32) + 0.5
    lz = (zs_ref[0, 0, :] - sz).astype(jnp.float32) + 0.5
    fx = intr_ref[0, 0]
    fy = intr_ref[0, 1]
    cx = intr_ref[0, 2]
    cy = intr_ref[0, 3]
    cnt = jnp.zeros(lx.shape, jnp.int32)
    for v in range(4):
        cam0 = lx * lm_ref[3 * v + 0, 0] + ly * lm_ref[3 * v + 0, 1] \
            + lz * lm_ref[3 * v + 0, 2] + lm_ref[3 * v + 0, 3]
        cam1 = lx * lm_ref[3 * v + 1, 0] + ly * lm_ref[3 * v + 1, 1] \
            + lz * lm_ref[3 * v + 1, 2] + lm_ref[3 * v + 1, 3]
        z = lx * lm_ref[3 * v + 2, 0] + ly * lm_ref[3 * v + 2, 1] \
            + lz * lm_ref[3 * v + 2, 2] + lm_ref[3 * v + 2, 3]
        zc = jnp.where(jnp.abs(z) < 1e-6, 1e-6, z)
        u = fx * cam0 / zc + cx
        vv = fy * cam1 / zc + cy
        valid = (z >= 0.1 / 0.05) & (z <= 4.0 / 0.05) \
            & (u >= 0.0) & (u < float(W)) & (vv >= 0.0) & (vv < float(H))
        ui = jnp.floor(jnp.clip(u, 0.0, float(W - 1))).astype(jnp.int32)
        vi = jnp.floor(jnp.clip(vv, 0.0, float(H - 1))).astype(jnp.int32)
        lin = v * HW + vi * W + ui
        lane = jax.lax.iota(jnp.int32, _BN2)
        pad_idx = 4 * HW + ((lane * 4 + v) & (_ZPAD - 1))
        pk_ref[0, v, :] = jnp.where(valid, lin, pad_idx)
        cnt = cnt + valid.astype(jnp.int32)
    cnt_ref[0, 0, :] = cnt
    rc_ref[0, 0, :] = 1.0 / (cnt.astype(jnp.float32) + 1e-4)


def _sc_body(feats_hbm, pk_hbm, rc_hbm, out_hbm, g, sem):
    def _step(iv, rv, ov):
        pltpu.async_copy(feats_hbm.at[iv.at[0, 0]], g, sem).wait()
        z16 = jax.lax.iota(jnp.int32, 16) * 0

        @pl.loop(0, _CB)
        def _row(r):
            rb = plsc.load_gather(rv, [z16, z16, z16 + r])
            for c in range(4):
                sl = pl.ds(c * 16, 16)
                a = g[r, sl] + g[_CB + r, sl]
                b = g[2 * _CB + r, sl] + g[3 * _CB + r, sl]
                ov[r, sl] = (a + b) * rb

    pltpu.emit_pipeline(
        _step,
        grid=(_NP // _CB,),
        in_specs=[
            pl.BlockSpec((1, 1, 4 * _CB), lambda i: (i, 0, 0)),
            pl.BlockSpec((1, 1, _CB), lambda i: (i, 0, 0)),
        ],
        out_specs=[pl.BlockSpec((_CB, 64), lambda i: (i, 0))],
        core_axis_name=("c", "s"),
        dimension_semantics=(pltpu.PARALLEL,),
    )(pk_hbm, rc_hbm, out_hbm)


def kernel(encoded_2d_features, coords, view_matrix, intrinsic_params):
    B, V, H, W, C = encoded_2d_features.shape
    N = coords.shape[0]
    HW = H * W
    pad = _NP - N

    xs = jnp.pad(coords[:, 1], (0, pad), constant_values=_BIG)
    ys = jnp.pad(coords[:, 2], (0, pad), constant_values=_BIG)
    zs = jnp.pad(coords[:, 3], (0, pad), constant_values=_BIG)
    xs3 = xs.reshape(_NBLK2, 1, _BN2)
    ys3 = ys.reshape(_NBLK2, 1, _BN2)
    zs3 = zs.reshape(_NBLK2, 1, _BN2)

    mx, my, mz = pl.pallas_call(
        _min_body,
        out_shape=[jax.ShapeDtypeStruct((1, 128), jnp.int32)] * 3,
    )(xs.reshape(_NP // 128, 128), ys.reshape(_NP // 128, 128),
      zs.reshape(_NP // 128, 128))
    sx = jnp.min(mx)
    sy = jnp.min(my)
    sz = jnp.min(mz)
    shift4 = jnp.stack([jnp.int32(0), sx, sy, sz]).reshape(1, 4)

    shift_f = jnp.stack([sx, sy, sz]).astype(jnp.float32)
    lview = view_matrix[0].at[:, :3, 3].add(-shift_f)
    lm = lview[:, :3, :].reshape(12, 4)
    intr = intrinsic_params.reshape(1, 4)

    blk = pl.BlockSpec((1, 1, _BN2), lambda i: (i, 0, 0))
    smem = pl.BlockSpec(memory_space=pltpu.SMEM)
    pk_raw, cnt, rc = pl.pallas_call(
        functools.partial(_proj_body, H=H, W=W, HW=HW),
        grid=(_NBLK2,),
        in_specs=[smem, smem, smem, blk, blk, blk],
        out_specs=[pl.BlockSpec((1, 4, _BN2), lambda i: (i, 0, 0)), blk, blk],
        out_shape=[
            jax.ShapeDtypeStruct((_NBLK2, 4, _BN2), jnp.int32),
            jax.ShapeDtypeStruct((_NBLK2, 1, _BN2), jnp.int32),
            jax.ShapeDtypeStruct((_NBLK2, 1, _BN2), jnp.float32),
        ],
    )(shift4, lm, intr, xs3, ys3, zs3)
    pk = pk_raw.reshape(_NBLK2, 4, _BN2 // _CB, _CB).transpose(0, 2, 1, 3) \
        .reshape(_NP // _CB, 1, 4 * _CB)
    rc2 = rc.reshape(_NP // _CB, 1, _CB)

    mesh = plsc.VectorSubcoreMesh(core_axis_name="c", subcore_axis_name="s")
    sc = pl.kernel(
        _sc_body,
        mesh=mesh,
        compiler_params=pltpu.CompilerParams(
            use_tc_tiling_on_sc=False, needs_layout_passes=False),
        out_type=jax.ShapeDtypeStruct((_NP, C), jnp.float32),
        scratch_types=[
            pltpu.VMEM((4 * _CB, C), jnp.float32),
            pltpu.SemaphoreType.DMA,
        ],
    )
    feats_tbl = jnp.concatenate(
        [encoded_2d_features.reshape(-1),
         jnp.zeros((_ZPAD * C,), jnp.float32)]).reshape(V * HW + _ZPAD, C)
    out_full = sc(feats_tbl, pk, rc2)

    return out_full[:N], cnt.reshape(_NP)[:N]


# speedup vs baseline: 12.5783x; 1.0568x over previous
import functools

import jax
import jax.numpy as jnp
from jax import lax
from jax.experimental import pallas as pl
from jax.experimental.pallas import tpu as pltpu
from jax.experimental.pallas import tpu_sc as plsc

_NW = 32
_CB = 125
_NCH = 25
_NP = _NW * _NCH * _CB
_NC = _NW * _NCH
_BN2 = 2000
_NBLK2 = _NP // _BN2
_ZPAD = 8192
_BIG = 1 << 29
_NPM = 100096


def _min_body(x_ref, y_ref, z_ref, ox_ref, oy_ref, oz_ref):
    ox_ref[...] = jnp.min(x_ref[...], axis=0, keepdims=True)
    oy_ref[...] = jnp.min(y_ref[...], axis=0, keepdims=True)
    oz_ref[...] = jnp.min(z_ref[...], axis=0, keepdims=True)


def _proj_body(shift_ref, lm_ref, intr_ref, xs_ref, ys_ref, zs_ref,
               pk_ref, cnt_ref, rc_ref, *, H, W, HW):
    sx = shift_ref[0, 1]
    sy = shift_ref[0, 2]
    sz = shift_ref[0, 3]
    lx = (xs_ref[0, 0, :] - sx).astype(jnp.float32) + 0.5
    ly = (ys_ref[0, 0, :] - sy).astype(jnp.float32) + 0.5
    lz = (zs_ref[0, 0, :] - sz).astype(jnp.float32) + 0.5
    fx = intr_ref[0, 0]
    fy = intr_ref[0, 1]
    cx = intr_ref[0, 2]
    cy = intr_ref[0, 3]
    cnt = jnp.zeros(lx.shape, jnp.int32)
    for v in range(4):
        cam0 = lx * lm_ref[3 * v + 0, 0] + ly * lm_ref[3 * v + 0, 1] \
            + lz * lm_ref[3 * v + 0, 2] + lm_ref[3 * v + 0, 3]
        cam1 = lx * lm_ref[3 * v + 1, 0] + ly * lm_ref[3 * v + 1, 1] \
            + lz * lm_ref[3 * v + 1, 2] + lm_ref[3 * v + 1, 3]
        z = lx * lm_ref[3 * v + 2, 0] + ly * lm_ref[3 * v + 2, 1] \
            + lz * lm_ref[3 * v + 2, 2] + lm_ref[3 * v + 2, 3]
        zc = jnp.where(jnp.abs(z) < 1e-6, 1e-6, z)
        u = fx * cam0 / zc + cx
        vv = fy * cam1 / zc + cy
        valid = (z >= 0.1 / 0.05) & (z <= 4.0 / 0.05) \
            & (u >= 0.0) & (u < float(W)) & (vv >= 0.0) & (vv < float(H))
        ui = jnp.floor(jnp.clip(u, 0.0, float(W - 1))).astype(jnp.int32)
        vi = jnp.floor(jnp.clip(vv, 0.0, float(H - 1))).astype(jnp.int32)
        lin = v * HW + vi * W + ui
        lane = jax.lax.iota(jnp.int32, _BN2)
        pad_idx = 4 * HW + ((lane * 4 + v) & (_ZPAD - 1))
        pk_ref[0, v, :] = jnp.where(valid, lin, pad_idx)
        cnt = cnt + valid.astype(jnp.int32)
    cnt_ref[0, 0, :] = cnt
    rc_ref[0, 0, :] = 1.0 / (cnt.astype(jnp.float32) + 1e-4)


def _sc_body(feats_hbm, pk_hbm, rc_hbm, out_hbm, g, sem):
    def _step(iv, rv, ov):
        pltpu.async_copy(feats_hbm.at[iv.at[0, 0]], g, sem).wait()
        z16 = jax.lax.iota(jnp.int32, 16) * 0

        @pl.loop(0, _CB)
        def _row(r):
            rb = plsc.load_gather(rv, [z16, z16, z16 + r])
            for c in range(4):
                sl = pl.ds(c * 16, 16)
                a = g[r, sl] + g[128 + r, sl]
                b = g[256 + r, sl] + g[384 + r, sl]
                ov[r, sl] = (a + b) * rb

    pltpu.emit_pipeline(
        _step,
        grid=(_NC,),
        in_specs=[
            pl.BlockSpec((1, 1, 512), lambda i: (i, 0, 0)),
            pl.BlockSpec((1, 1, 128), lambda i: (i, 0, 0)),
        ],
        out_specs=[pl.BlockSpec((_CB, 64), lambda i: (i, 0))],
        core_axis_name=("c", "s"),
        dimension_semantics=(pltpu.PARALLEL,),
    )(pk_hbm, rc_hbm, out_hbm)


def kernel(encoded_2d_features, coords, view_matrix, intrinsic_params):
    B, V, H, W, C = encoded_2d_features.shape
    N = coords.shape[0]
    HW = H * W

    xs = coords[:, 1]
    ys = coords[:, 2]
    zs = coords[:, 3]
    xs3 = xs.reshape(_NBLK2, 1, _BN2)
    ys3 = ys.reshape(_NBLK2, 1, _BN2)
    zs3 = zs.reshape(_NBLK2, 1, _BN2)

    padm = _NPM - N
    mx, my, mz = pl.pallas_call(
        _min_body,
        out_shape=[jax.ShapeDtypeStruct((1, 128), jnp.int32)] * 3,
    )(jnp.pad(xs, (0, padm), constant_values=_BIG).reshape(_NPM // 128, 128),
      jnp.pad(ys, (0, padm), constant_values=_BIG).reshape(_NPM // 128, 128),
      jnp.pad(zs, (0, padm), constant_values=_BIG).reshape(_NPM // 128, 128))
    sx = jnp.min(mx)
    sy = jnp.min(my)
    sz = jnp.min(mz)
    shift4 = jnp.stack([jnp.int32(0), sx, sy, sz]).reshape(1, 4)

    shift_f = jnp.stack([sx, sy, sz]).astype(jnp.float32)
    lview = view_matrix[0].at[:, :3, 3].add(-shift_f)
    lm = lview[:, :3, :].reshape(12, 4)
    intr = intrinsic_params.reshape(1, 4)

    blk = pl.BlockSpec((1, 1, _BN2), lambda i: (i, 0, 0))
    smem = pl.BlockSpec(memory_space=pltpu.SMEM)
    pk_raw, cnt, rc = pl.pallas_call(
        functools.partial(_proj_body, H=H, W=W, HW=HW),
        grid=(_NBLK2,),
        in_specs=[smem, smem, smem, blk, blk, blk],
        out_specs=[pl.BlockSpec((1, 4, _BN2), lambda i: (i, 0, 0)), blk, blk],
        out_shape=[
            jax.ShapeDtypeStruct((_NBLK2, 4, _BN2), jnp.int32),
            jax.ShapeDtypeStruct((_NBLK2, 1, _BN2), jnp.int32),
            jax.ShapeDtypeStruct((_NBLK2, 1, _BN2), jnp.float32),
        ],
    )(shift4, lm, intr, xs3, ys3, zs3)
    pk4 = pk_raw.reshape(_NBLK2, 4, _BN2 // _CB, _CB).transpose(0, 2, 1, 3) \
        .reshape(_NC, 4, _CB)
    padcols = 4 * HW + (jnp.arange(_NC * 4 * 3, dtype=jnp.int32)
                        .reshape(_NC, 4, 3) & (_ZPAD - 1))
    pk = jnp.concatenate([pk4, padcols], axis=2).reshape(_NC, 1, 512)
    rc2 = jnp.pad(rc.reshape(_NC, _CB), ((0, 0), (0, 3))).reshape(_NC, 1, 128)

    mesh = plsc.VectorSubcoreMesh(core_axis_name="c", subcore_axis_name="s")
    sc = pl.kernel(
        _sc_body,
        mesh=mesh,
        compiler_params=pltpu.CompilerParams(
            use_tc_tiling_on_sc=False, needs_layout_passes=False),
        out_type=jax.ShapeDtypeStruct((_NP, C), jnp.float32),
        scratch_types=[
            pltpu.VMEM((512, C), jnp.float32),
            pltpu.SemaphoreType.DMA,
        ],
    )
    feats_tbl = jnp.concatenate(
        [encoded_2d_features.reshape(-1),
         jnp.zeros((_ZPAD * C,), jnp.float32)]).reshape(V * HW + _ZPAD, C)
    out_full = sc(feats_tbl, pk, rc2)

    return out_full, cnt.reshape(_NP)
